# baseline (speedup 1.0000x reference)
import functools
import math

import jax
import jax.numpy as jnp
from jax import lax
from jax.experimental import pallas as pl
from jax.experimental.pallas import tpu as pltpu
from jax.experimental.pallas import tpu_sc as plsc

D_MODEL = 1024
_SCALE = math.sqrt(D_MODEL)
_LANES = 16
_NC = 2
_NS = 16
_NW = _NC * _NS
_C = 8
_NB = 8


def _make_sc_kernel(B: int, n_cols: int):
    rpw = B // _NW
    nch = rpw // _C
    n_outer = nch // _NB
    wpr = n_cols // rpw
    mesh = plsc.VectorSubcoreMesh(core_axis_name="c", subcore_axis_name="s")

    @functools.partial(
        pl.kernel,
        mesh=mesh,
        out_type=jax.ShapeDtypeStruct((B, D_MODEL), jnp.float32),
        scratch_types=[
            pltpu.VMEM((rpw,), jnp.int32),
        ]
        + [pltpu.VMEM((_C, D_MODEL), jnp.float32)] * _NB
        + [pltpu.SemaphoreType.DMA] * (2 * _NB),
    )
    def gather_scale(x_hbm, table_hbm, out_hbm, idx_v, *rest):
        bufs = rest[:_NB]
        gsems = rest[_NB:2 * _NB]
        ssems = rest[2 * _NB:]
        wid = lax.axis_index("s") * _NC + lax.axis_index("c")
        base = wid * rpw
        pltpu.sync_copy(
            x_hbm.at[wid // wpr, pl.ds((wid % wpr) * rpw, rpw)],
            idx_v)

        def start_gather(k, b):
            pltpu.async_copy(table_hbm.at[idx_v.at[pl.ds(k * _C, _C)]], bufs[b], gsems[b])

        def wait_gather(b):
            pltpu.make_async_copy(
                table_hbm.at[idx_v.at[pl.ds(0, _C)]], bufs[b], gsems[b]).wait()

        def start_store(k, b):
            pltpu.async_copy(
                bufs[b], out_hbm.at[pl.ds(base + k * _C, _C)], ssems[b])

        def wait_store(b):
            pltpu.make_async_copy(
                bufs[b], out_hbm.at[pl.ds(0, _C)], ssems[b]).wait()

        def scale(b):
            buf = bufs[b]

            def slice_body(j, c2):
                r = j >> 6
                c = pl.multiple_of((j & 63) * _LANES, _LANES)
                buf[r, pl.ds(c, _LANES)] = buf[r, pl.ds(c, _LANES)] * _SCALE
                return c2

            lax.fori_loop(0, _C * (D_MODEL // _LANES), slice_body, 0)

        for b in range(_NB):
            start_gather(b, b)

        def outer(g, carry):
            for b in range(_NB):
                k = g * _NB + b
                wait_gather(b)
                scale(b)
                start_store(k, b)
                bp = (b - 1) % _NB
                cond = (g >= 1) if b == 0 else (g < n_outer - 1)

                @pl.when(cond)
                def _():
                    wait_store(bp)
                    start_gather(k + _NB - 1, bp)
            return carry

        lax.fori_loop(0, n_outer, outer, 0)
        for b in range(_NB):
            wait_store(b)

    return gather_scale


def kernel(x, table):
    B = x.size
    rpw = B // _NW
    assert x.shape[-1] % rpw == 0
    out = _make_sc_kernel(B, x.shape[-1])(x.astype(jnp.int32), table)
    return out.reshape(x.shape + (D_MODEL,))


# speedup vs baseline: 3.1846x; 3.1846x over previous
import functools
import math

import jax
import jax.numpy as jnp
from jax import lax
from jax.experimental import pallas as pl
from jax.experimental.pallas import tpu as pltpu
from jax.experimental.pallas import tpu_sc as plsc

D_MODEL = 1024
_SCALE = math.sqrt(D_MODEL)
_LANES = 16
_NC = 2
_NS = 16
_NW = _NC * _NS
_C = 8
_NB = 8


def _make_sc_kernel(B: int, n_cols: int):
    rpw = B // _NW
    nch = rpw // _C
    n_outer = nch // _NB
    wpr = n_cols // rpw
    mesh = plsc.VectorSubcoreMesh(core_axis_name="c", subcore_axis_name="s")

    @functools.partial(
        pl.kernel,
        mesh=mesh,
        out_type=jax.ShapeDtypeStruct((B, D_MODEL), jnp.float32),
        scratch_types=[
            pltpu.VMEM((rpw,), jnp.int32),
        ]
        + [pltpu.VMEM((_C, D_MODEL), jnp.float32)] * _NB
        + [pltpu.SemaphoreType.DMA] * (2 * _NB),
    )
    def gather_scale(x_hbm, table_hbm, out_hbm, idx_v, *rest):
        bufs = rest[:_NB]
        gsems = rest[_NB:2 * _NB]
        ssems = rest[2 * _NB:]
        wid = lax.axis_index("s") * _NC + lax.axis_index("c")
        base = wid * rpw
        pltpu.sync_copy(
            x_hbm.at[wid // wpr, pl.ds((wid % wpr) * rpw, rpw)],
            idx_v)

        def start_gather(k, b):
            pltpu.async_copy(table_hbm.at[idx_v.at[pl.ds(k * _C, _C)]], bufs[b], gsems[b])

        def wait_gather(b):
            pltpu.make_async_copy(
                table_hbm.at[idx_v.at[pl.ds(0, _C)]], bufs[b], gsems[b]).wait()

        def start_store(k, b):
            pltpu.async_copy(
                bufs[b], out_hbm.at[pl.ds(base + k * _C, _C)], ssems[b])

        def wait_store(b):
            pltpu.make_async_copy(
                bufs[b], out_hbm.at[pl.ds(0, _C)], ssems[b]).wait()

        def scale(b):
            buf = bufs[b]

            def row_body(r, c2):
                for j in range(D_MODEL // _LANES):
                    sl = pl.ds(j * _LANES, _LANES)
                    buf[r, sl] = buf[r, sl] * _SCALE
                return c2

            lax.fori_loop(0, _C, row_body, 0)

        for b in range(_NB):
            start_gather(b, b)

        def outer(g, carry):
            for b in range(_NB):
                k = g * _NB + b
                wait_gather(b)
                scale(b)
                start_store(k, b)
                bp = (b - 1) % _NB
                cond = (g >= 1) if b == 0 else (g < n_outer - 1)

                @pl.when(cond)
                def _():
                    wait_store(bp)
                    start_gather(k + _NB - 1, bp)
            return carry

        lax.fori_loop(0, n_outer, outer, 0)
        for b in range(_NB):
            wait_store(b)

    return gather_scale


def kernel(x, table):
    B = x.size
    rpw = B // _NW
    assert x.shape[-1] % rpw == 0
    out = _make_sc_kernel(B, x.shape[-1])(x.astype(jnp.int32), table)
    return out.reshape(x.shape + (D_MODEL,))
